# Initial kernel scaffold; baseline (speedup 1.0000x reference)
import jax
import jax.numpy as jnp
from jax.experimental import pallas as pl


def kernel(input, neighbor_idx, weight, bias):
    raise NotImplementedError("write your pallas kernel here")


# speedup vs baseline: 13.7785x; 13.7785x over previous
import functools

import jax
import jax.numpy as jnp
from jax import lax
from jax.experimental import pallas as pl
from jax.experimental.pallas import tpu as pltpu
from jax.experimental.pallas import tpu_sc as plsc

N = 100000
K = 27
C = 16

CB = 128
NPAD = 100096
NCHUNK = NPAD // CB
NW = 32
SC_ITERS = -(-NCHUNK // NW)

BN = 2000


def _mm_body(x_ref, w_ref, o_ref):
    o_ref[...] = jnp.dot(x_ref[...], w_ref[...],
                         preferred_element_type=jnp.float32)


def _tc_matmul(x, wcat):
    return pl.pallas_call(
        _mm_body,
        grid=(N // BN,),
        in_specs=[pl.BlockSpec((BN, C), lambda i: (i, 0)),
                  pl.BlockSpec((C, K * C), lambda i: (0, 0))],
        out_specs=pl.BlockSpec((BN, K * C), lambda i: (i, 0)),
        out_shape=jax.ShapeDtypeStruct((N, K * C), jnp.float32),
    )(x, wcat)


_MESH = plsc.VectorSubcoreMesh(core_axis_name="c", subcore_axis_name="s")


@functools.partial(
    pl.kernel,
    out_type=jax.ShapeDtypeStruct((NPAD, C), jnp.float32),
    mesh=_MESH,
    scratch_types=[
        pltpu.VMEM((K, CB), jnp.int32),
        pltpu.VMEM((K * CB, C), jnp.float32),
        pltpu.VMEM((CB, C), jnp.float32),
        pltpu.VMEM((C,), jnp.float32),
        pltpu.SemaphoreType.DMA,
    ],
    compiler_params=pltpu.CompilerParams(use_tc_tiling_on_sc=False),
)
def _sc_gather_reduce(ztab, idxt, bias_h, out, idx_v, rows_v, out_v,
                      bias_v, sem):
    w = lax.axis_index("s") * 2 + lax.axis_index("c")
    pltpu.sync_copy(bias_h, bias_v)

    def chunk_body(i, carry):
        c = i * NW + w

        @pl.when(c < NCHUNK)
        def _():
            pltpu.sync_copy(idxt.at[c], idx_v)

            def krow(k, carry2):
                for t in range(CB // 16):
                    sl = pl.ds(t * 16, 16)
                    idx_v[k, sl] = idx_v[k, sl] * K + k
                return carry2

            lax.fori_loop(0, K, krow, 0)

            def fire(k, carry2):
                pltpu.async_copy(ztab.at[idx_v.at[k]],
                                 rows_v.at[pl.ds(k * CB, CB)], sem)
                return carry2

            lax.fori_loop(0, K, fire, 0)
            pltpu.make_async_copy(ztab.at[pl.ds(0, K * CB)], rows_v,
                                  sem).wait()

            def node(j, carry2):
                acc = bias_v[...] + rows_v[j, :]
                for k in range(1, K):
                    acc = acc + rows_v[k * CB + j, :]
                out_v[j, :] = acc
                return carry2

            lax.fori_loop(0, CB, node, 0)
            pltpu.sync_copy(out_v, out.at[pl.ds(c * CB, CB)])

        return carry

    lax.fori_loop(0, SC_ITERS, chunk_body, 0)


def kernel(input, neighbor_idx, weight, bias):
    wcat = jnp.transpose(weight, (1, 0, 2)).reshape(C, K * C)
    z = _tc_matmul(input, wcat)
    ztab = z.reshape(N * K, C)
    nid = jnp.pad(neighbor_idx, ((0, NPAD - N), (0, 0)))
    idxt = nid.reshape(NCHUNK, CB, K).transpose(0, 2, 1)
    out = _sc_gather_reduce(ztab, idxt, bias)
    return out[:N]


# speedup vs baseline: 15.6578x; 1.1364x over previous
import functools

import jax
import jax.numpy as jnp
from jax import lax
from jax.experimental import pallas as pl
from jax.experimental.pallas import tpu as pltpu
from jax.experimental.pallas import tpu_sc as plsc

N = 100000
K = 27
C = 16

CB = 128
NPAD = 100096
NCHUNK = NPAD // CB
NW = 32
ROWS = K * CB
LASTV = N - (NCHUNK - 1) * CB
BASE_CNT = NCHUNK // NW
EXTRA = NCHUNK % NW
MAXC = BASE_CNT + 1

BN = 2000


def _mm_body(x_ref, w_ref, o_ref):
    o_ref[...] = jnp.dot(x_ref[...], w_ref[...],
                         preferred_element_type=jnp.float32)


def _tc_matmul(x, wcat):
    return pl.pallas_call(
        _mm_body,
        grid=(N // BN,),
        in_specs=[pl.BlockSpec((BN, C), lambda i: (i, 0)),
                  pl.BlockSpec((C, K * C), lambda i: (0, 0))],
        out_specs=pl.BlockSpec((BN, K * C), lambda i: (i, 0)),
        out_shape=jax.ShapeDtypeStruct((N, K * C), jnp.float32),
    )(x, wcat)


_MESH = plsc.VectorSubcoreMesh(core_axis_name="c", subcore_axis_name="s")


@functools.partial(
    pl.kernel,
    out_type=jax.ShapeDtypeStruct((N, C), jnp.float32),
    mesh=_MESH,
    scratch_types=[
        pltpu.VMEM((K, CB), jnp.int32),
        pltpu.VMEM((K, CB), jnp.int32),
        pltpu.VMEM((ROWS, C), jnp.float32),
        pltpu.VMEM((ROWS, C), jnp.float32),
        pltpu.VMEM((CB, C), jnp.float32),
        pltpu.VMEM((C,), jnp.float32),
        pltpu.SemaphoreType.DMA,
        pltpu.SemaphoreType.DMA,
    ],
    compiler_params=pltpu.CompilerParams(use_tc_tiling_on_sc=False),
)
def _sc_gather_reduce(ztab, idxn, bias_h, out, idx0, idx1, rows0, rows1,
                      out_v, bias_v, sem0, sem1):
    w = lax.axis_index("s") * 2 + lax.axis_index("c")
    start = w * BASE_CNT + jnp.minimum(w, EXTRA)
    cnt = BASE_CNT + jnp.where(w < EXTRA, 1, 0)
    pltpu.sync_copy(bias_h, bias_v)
    iot = lax.iota(jnp.int32, 16)

    def prep(i, idx_v, rows_v, sem):
        c = start + i
        pltpu.sync_copy(idxn.at[c], idx_v)

        def krow(g, carry):
            for t in range(CB // 16):
                sl = pl.ds(t * 16, 16)
                p = g * CB + t * 16 + iot
                idx_v[g, sl] = idx_v[g, sl] * K + lax.rem(p, K)
            return carry

        lax.fori_loop(0, K, krow, 0)

        def fire(g, carry):
            pltpu.async_copy(ztab.at[idx_v.at[g]],
                             rows_v.at[pl.ds(g * CB, CB)], sem)
            return carry

        lax.fori_loop(0, K, fire, 0)

    def consume(i, rows_v, sem):
        c = start + i
        pltpu.make_async_copy(ztab.at[pl.ds(0, ROWS)], rows_v, sem).wait()

        def node(j, carry):
            base = j * K
            acc = bias_v[...] + rows_v[base, :]
            for k in range(1, K):
                acc = acc + rows_v[base + k, :]
            out_v[j, :] = acc
            return carry

        lax.fori_loop(0, CB, node, 0)

        @pl.when(c < NCHUNK - 1)
        def _():
            pltpu.sync_copy(out_v, out.at[pl.ds(c * CB, CB)])

        @pl.when(c == NCHUNK - 1)
        def _():
            pltpu.sync_copy(out_v.at[pl.ds(0, LASTV)],
                            out.at[pl.ds(c * CB, LASTV)])

    prep(0, idx0, rows0, sem0)

    def pipe(t, carry):
        i0 = t * 2
        i1 = i0 + 1

        @pl.when(i1 < cnt)
        def _():
            prep(i1, idx1, rows1, sem1)

        @pl.when(i0 < cnt)
        def _():
            consume(i0, rows0, sem0)

        @pl.when(i1 + 1 < cnt)
        def _():
            prep(i1 + 1, idx0, rows0, sem0)

        @pl.when(i1 < cnt)
        def _():
            consume(i1, rows1, sem1)

        return carry

    lax.fori_loop(0, (MAXC + 1) // 2, pipe, 0)


def kernel(input, neighbor_idx, weight, bias):
    wcat = jnp.transpose(weight, (1, 0, 2)).reshape(C, K * C)
    z = _tc_matmul(input, wcat)
    ztab = z.reshape(N * K, C)
    nid = jnp.pad(neighbor_idx, ((0, NPAD - N), (0, 0)))
    idxn = nid.reshape(NCHUNK, K, CB)
    return _sc_gather_reduce(ztab, idxn, bias)


# speedup vs baseline: 19.5305x; 1.2473x over previous
import functools

import jax
import jax.numpy as jnp
from jax import lax
from jax.experimental import pallas as pl
from jax.experimental.pallas import tpu as pltpu
from jax.experimental.pallas import tpu_sc as plsc

N = 100000
K = 27
C = 16

CB = 128
NPAD = 100096
NCHUNK = NPAD // CB
NW = 32
ROWS = K * CB
LASTV = N - (NCHUNK - 1) * CB
BASE_CNT = NCHUNK // NW
EXTRA = NCHUNK % NW
MAXC = BASE_CNT + 1

K2 = 32
BN = 2000


def _mm_body(x_ref, w_ref, o_ref):
    o_ref[...] = jnp.dot(x_ref[...], w_ref[...],
                         preferred_element_type=jnp.float32)


def _tc_matmul(x, wcat):
    return pl.pallas_call(
        _mm_body,
        grid=(N // BN,),
        in_specs=[pl.BlockSpec((BN, C), lambda i: (i, 0)),
                  pl.BlockSpec((C, K2 * C), lambda i: (0, 0))],
        out_specs=pl.BlockSpec((BN, K2 * C), lambda i: (i, 0)),
        out_shape=jax.ShapeDtypeStruct((N, K2 * C), jnp.float32),
    )(x, wcat)


_MESH = plsc.VectorSubcoreMesh(core_axis_name="c", subcore_axis_name="s")


@functools.partial(
    pl.kernel,
    out_type=jax.ShapeDtypeStruct((N, C), jnp.float32),
    mesh=_MESH,
    scratch_types=[
        pltpu.VMEM((K, CB), jnp.int32),
        pltpu.VMEM((K, CB), jnp.int32),
        pltpu.VMEM((ROWS, C), jnp.float32),
        pltpu.VMEM((ROWS, C), jnp.float32),
        pltpu.VMEM((CB, C), jnp.float32),
        pltpu.VMEM((C,), jnp.float32),
        pltpu.SemaphoreType.DMA,
        pltpu.SemaphoreType.DMA,
    ],
    compiler_params=pltpu.CompilerParams(use_tc_tiling_on_sc=False),
)
def _sc_gather_reduce(ztab, idxn, bias_h, out, idx0, idx1, rows0, rows1,
                      out_v, bias_v, sem0, sem1):
    w = lax.axis_index("s") * 2 + lax.axis_index("c")
    start = w * BASE_CNT + jnp.minimum(w, EXTRA)
    cnt = BASE_CNT + jnp.where(w < EXTRA, 1, 0)
    pltpu.sync_copy(bias_h, bias_v)
    iot = lax.iota(jnp.int32, 16)

    def prep(i, idx_v, rows_v, sem):
        c = start + i
        pltpu.sync_copy(idxn.at[c], idx_v)

        def krow(g, carry):
            for t in range(CB // 16):
                sl = pl.ds(t * 16, 16)
                p = g * CB + t * 16 + iot
                idx_v[g, sl] = lax.shift_left(idx_v[g, sl], 5) + lax.rem(p, K)
            return carry

        lax.fori_loop(0, K, krow, 0)

        def fire(g, carry):
            pltpu.async_copy(ztab.at[idx_v.at[g]],
                             rows_v.at[pl.ds(g * CB, CB)], sem)
            return carry

        lax.fori_loop(0, K, fire, 0)

    def consume(i, rows_v, sem):
        c = start + i
        pltpu.make_async_copy(ztab.at[pl.ds(0, ROWS)], rows_v, sem).wait()

        def node(j, carry):
            base = j * K
            acc = bias_v[...] + rows_v[base, :]
            for k in range(1, K):
                acc = acc + rows_v[base + k, :]
            out_v[j, :] = acc
            return carry

        lax.fori_loop(0, CB, node, 0)

        @pl.when(c < NCHUNK - 1)
        def _():
            pltpu.sync_copy(out_v, out.at[pl.ds(c * CB, CB)])

        @pl.when(c == NCHUNK - 1)
        def _():
            pltpu.sync_copy(out_v.at[pl.ds(0, LASTV)],
                            out.at[pl.ds(c * CB, LASTV)])

    prep(0, idx0, rows0, sem0)

    def pipe(t, carry):
        i0 = t * 2
        i1 = i0 + 1

        @pl.when(i1 < cnt)
        def _():
            prep(i1, idx1, rows1, sem1)

        @pl.when(i0 < cnt)
        def _():
            consume(i0, rows0, sem0)

        @pl.when(i1 + 1 < cnt)
        def _():
            prep(i1 + 1, idx0, rows0, sem0)

        @pl.when(i1 < cnt)
        def _():
            consume(i1, rows1, sem1)

        return carry

    lax.fori_loop(0, (MAXC + 1) // 2, pipe, 0)


def kernel(input, neighbor_idx, weight, bias):
    wpad = jnp.pad(weight, ((0, K2 - K), (0, 0), (0, 0)))
    wcat = jnp.transpose(wpad, (1, 0, 2)).reshape(C, K2 * C)
    z = _tc_matmul(input, wcat)
    ztab = z.reshape(N * K2, C)
    nid = jnp.pad(neighbor_idx, ((0, NPAD - N), (0, 0)))
    idxn = nid.reshape(NCHUNK, K, CB)
    return _sc_gather_reduce(ztab, idxn, bias)


# speedup vs baseline: 20.7371x; 1.0618x over previous
import functools

import jax
import jax.numpy as jnp
from jax import lax
from jax.experimental import pallas as pl
from jax.experimental.pallas import tpu as pltpu
from jax.experimental.pallas import tpu_sc as plsc

N = 100000
K = 27
C = 16

CB = 128
NPAD = 100096
NCHUNK = NPAD // CB
NW = 32
ROWS = K * CB
LASTV = N - (NCHUNK - 1) * CB
BASE_CNT = NCHUNK // NW
EXTRA = NCHUNK % NW
MAXC = BASE_CNT + 1

K2 = 32
NT = 4
PLANE = N * 8
BN = 2000


def _mm_body(x_ref, w_ref, o_ref):
    o_ref[...] = jnp.dot(x_ref[...], w_ref[...],
                         preferred_element_type=jnp.float32)[None]


def _tc_matmul(x, wcat):
    return pl.pallas_call(
        _mm_body,
        grid=(N // BN, NT),
        in_specs=[pl.BlockSpec((BN, C), lambda i, t: (i, 0)),
                  pl.BlockSpec((C, 8 * C), lambda i, t: (0, t))],
        out_specs=pl.BlockSpec((1, BN, 8 * C), lambda i, t: (t, i, 0)),
        out_shape=jax.ShapeDtypeStruct((NT, N, 8 * C), jnp.float32),
    )(x, wcat)


_MESH = plsc.VectorSubcoreMesh(core_axis_name="c", subcore_axis_name="s")


@functools.partial(
    pl.kernel,
    out_type=jax.ShapeDtypeStruct((N, C), jnp.float32),
    mesh=_MESH,
    scratch_types=[
        pltpu.VMEM((K, CB), jnp.int32),
        pltpu.VMEM((K, CB), jnp.int32),
        pltpu.VMEM((ROWS, C), jnp.float32),
        pltpu.VMEM((ROWS, C), jnp.float32),
        pltpu.VMEM((CB, C), jnp.float32),
        pltpu.VMEM((C,), jnp.float32),
        pltpu.SemaphoreType.DMA,
        pltpu.SemaphoreType.DMA,
    ],
    compiler_params=pltpu.CompilerParams(use_tc_tiling_on_sc=False),
)
def _sc_gather_reduce(ztab, idxn, bias_h, out, idx0, idx1, rows0, rows1,
                      out_v, bias_v, sem0, sem1):
    w = lax.axis_index("s") * 2 + lax.axis_index("c")
    start = w * BASE_CNT + jnp.minimum(w, EXTRA)
    cnt = BASE_CNT + jnp.where(w < EXTRA, 1, 0)
    pltpu.sync_copy(bias_h, bias_v)
    iot = lax.iota(jnp.int32, 16)

    def prep(i, idx_v, rows_v, sem):
        c = start + i
        pltpu.sync_copy(idxn.at[c], idx_v)

        def krow(g, carry):
            for t in range(CB // 16):
                sl = pl.ds(t * 16, 16)
                p = g * CB + t * 16 + iot
                k = lax.rem(p, K)
                idx_v[g, sl] = (lax.shift_left(idx_v[g, sl], 3)
                                + lax.shift_right_logical(k, 3) * PLANE
                                + lax.bitwise_and(k, 7))
            return carry

        lax.fori_loop(0, K, krow, 0)

        def fire(g, carry):
            pltpu.async_copy(ztab.at[idx_v.at[g]],
                             rows_v.at[pl.ds(g * CB, CB)], sem)
            return carry

        lax.fori_loop(0, K, fire, 0)

    def consume(i, rows_v, sem):
        c = start + i
        pltpu.make_async_copy(ztab.at[pl.ds(0, ROWS)], rows_v, sem).wait()

        def node(j, carry):
            base = j * K
            acc = bias_v[...] + rows_v[base, :]
            for k in range(1, K):
                acc = acc + rows_v[base + k, :]
            out_v[j, :] = acc
            return carry

        lax.fori_loop(0, CB, node, 0)

        @pl.when(c < NCHUNK - 1)
        def _():
            pltpu.sync_copy(out_v, out.at[pl.ds(c * CB, CB)])

        @pl.when(c == NCHUNK - 1)
        def _():
            pltpu.sync_copy(out_v.at[pl.ds(0, LASTV)],
                            out.at[pl.ds(c * CB, LASTV)])

    prep(0, idx0, rows0, sem0)

    def pipe(t, carry):
        i0 = t * 2
        i1 = i0 + 1

        @pl.when(i1 < cnt)
        def _():
            prep(i1, idx1, rows1, sem1)

        @pl.when(i0 < cnt)
        def _():
            consume(i0, rows0, sem0)

        @pl.when(i1 + 1 < cnt)
        def _():
            prep(i1 + 1, idx0, rows0, sem0)

        @pl.when(i1 < cnt)
        def _():
            consume(i1, rows1, sem1)

        return carry

    lax.fori_loop(0, (MAXC + 1) // 2, pipe, 0)


def kernel(input, neighbor_idx, weight, bias):
    wpad = jnp.pad(weight, ((0, K2 - K), (0, 0), (0, 0)))
    wcat = jnp.transpose(wpad, (1, 0, 2)).reshape(C, K2 * C)
    z3 = _tc_matmul(input, wcat)
    ztab = z3.reshape(N * K2, C)
    nid = jnp.pad(neighbor_idx, ((0, NPAD - N), (0, 0)))
    idxn = nid.reshape(NCHUNK, K, CB)
    return _sc_gather_reduce(ztab, idxn, bias)
